# baseline (speedup 1.0000x reference)
import functools

import jax
import jax.numpy as jnp
from jax.experimental import pallas as pl
from jax.experimental.pallas import tpu as pltpu


def _pass1_kernel(x_ref, wg_ref, gum_ref, tok_ref, stats_ref, carry_ref, *,
                  nblocks, bt, ne):
    i = pl.program_id(0)

    @pl.when(i == 0)
    def _init():
        carry_ref[...] = jnp.zeros_like(carry_ref)

    x = x_ref[...]
    wg = wg_ref[...]
    logits = jax.lax.dot_general(
        x, wg, dimension_numbers=(((1,), (1,)), ((), ())),
        preferred_element_type=jnp.float32)

    m = jnp.max(logits, axis=1, keepdims=True)
    p = jnp.exp(logits - m)
    gates = p / jnp.sum(p, axis=1, keepdims=True)

    iota_e = jax.lax.broadcasted_iota(jnp.int32, (bt, ne), 1)

    gmax = jnp.max(gates, axis=1, keepdims=True)
    idx1 = jnp.min(jnp.where(gates == gmax, iota_e, ne), axis=1)
    mask1 = iota_e == idx1[:, None]
    mask1_f = mask1.astype(jnp.float32)

    lw = logits + gum_ref[...]
    lx = jnp.where(mask1, -jnp.inf, lw)
    m2 = jnp.max(lx, axis=1, keepdims=True)
    idx2 = jnp.min(jnp.where(lx == m2, iota_e, ne), axis=1)
    mask2 = iota_e == idx2[:, None]
    mask2_f = mask2.astype(jnp.float32)

    r = jax.lax.broadcasted_iota(jnp.int32, (bt, bt), 0)
    c = jax.lax.broadcasted_iota(jnp.int32, (bt, bt), 1)
    tri = (r >= c).astype(jnp.float32)
    c1 = jax.lax.dot_general(tri, mask1_f,
                             dimension_numbers=(((1,), (0,)), ((), ())),
                             preferred_element_type=jnp.float32)
    c2 = jax.lax.dot_general(tri, mask2_f,
                             dimension_numbers=(((1,), (0,)), ((), ())),
                             preferred_element_type=jnp.float32)

    carry1 = carry_ref[0:1, :]
    carry2 = carry_ref[1:2, :]

    loc1_s = jnp.sum((c1 - 1.0 + carry1) * mask1_f, axis=1)
    loc2p_s = jnp.sum((c2 - 1.0 + carry2) * mask2_f, axis=1)

    g1 = jnp.sum(gates * mask1_f, axis=1)
    g2 = jnp.sum(gates * mask2_f, axis=1)

    carry_ref[0:1, :] = carry1 + jnp.sum(mask1_f, axis=0, keepdims=True)
    carry_ref[1:2, :] = carry2 + jnp.sum(mask2_f, axis=0, keepdims=True)
    carry_ref[2:3, :] = carry_ref[2:3, :] + jnp.sum(gates, axis=0,
                                                    keepdims=True)

    e1f = idx1.astype(jnp.float32)
    e2f = idx2.astype(jnp.float32)
    zeros = jnp.zeros((bt,), jnp.float32)
    tok = jnp.stack([e1f, e2f, g1, g2, loc1_s, loc2p_s, zeros, zeros], axis=1)
    tok_ref[...] = tok

    @pl.when(i == nblocks - 1)
    def _emit_stats():
        stats_ref[...] = carry_ref[...]


def _pass2_kernel(tok_ref, stats_ref, comb_ref, disp_ref, laux_ref, *,
                  bt, ne, cap, ntok):
    i = pl.program_id(0)
    t = tok_ref[...]
    e1 = t[:, 0]
    e2 = t[:, 1]
    g1 = t[:, 2]
    g2 = t[:, 3]
    loc1 = t[:, 4]
    loc2p = t[:, 5]

    cnt1 = stats_ref[0:1, :]
    iota_e = jax.lax.broadcasted_iota(jnp.int32, (bt, ne), 1)
    oh2 = (iota_e == e2.astype(jnp.int32)[:, None]).astype(jnp.float32)
    loc2 = loc2p + jnp.sum(oh2 * cnt1, axis=1)

    capf = jnp.float32(cap)
    keep1 = loc1 < capf
    keep2 = loc2 < capf
    g1e = jnp.where(keep1, g1, 0.0)
    g2e = jnp.where(keep2, g2, 0.0)
    denom = jnp.maximum(g1e + g2e, jnp.finfo(jnp.float32).eps)
    g1n = g1e / denom
    g2n = g2e / denom

    loc1s = jnp.where(keep1, loc1, 0.0)
    loc2s = jnp.where(keep2, loc2, 0.0)

    oh1 = iota_e == e1.astype(jnp.int32)[:, None]
    iota_c = jax.lax.broadcasted_iota(jnp.int32, (bt, cap), 1)
    l1 = (iota_c == loc1s.astype(jnp.int32)[:, None]).astype(jnp.float32)
    l2 = (iota_c == loc2s.astype(jnp.int32)[:, None]).astype(jnp.float32)
    g1mat = jnp.where(oh1, g1n[:, None], 0.0)
    g2mat = jnp.where(oh2 != 0.0, g2n[:, None], 0.0)
    comb = g1mat[:, :, None] * l1[:, None, :] + g2mat[:, :, None] * l2[:, None, :]
    comb_ref[...] = comb
    disp_ref[...] = comb != 0.0

    @pl.when(i == 0)
    def _emit_laux():
        me = stats_ref[2:3, :] / jnp.float32(ntok)
        ce = stats_ref[0:1, :] / jnp.float32(ntok)
        laux_ref[0, 0] = jnp.sum(me * ce) * jnp.float32(ne)


def kernel(input, wg):
    ntok, dim = input.shape
    ne = wg.shape[0]
    cap = 2 * ntok // ne
    bt = 256
    nblocks = ntok // bt

    gumbel = jax.random.gumbel(jax.random.key(42), (ntok, ne),
                               dtype=jnp.float32)

    tok, stats = pl.pallas_call(
        functools.partial(_pass1_kernel, nblocks=nblocks, bt=bt, ne=ne),
        grid=(nblocks,),
        in_specs=[
            pl.BlockSpec((bt, dim), lambda i: (i, 0)),
            pl.BlockSpec((ne, dim), lambda i: (0, 0)),
            pl.BlockSpec((bt, ne), lambda i: (i, 0)),
        ],
        out_specs=[
            pl.BlockSpec((bt, 8), lambda i: (i, 0)),
            pl.BlockSpec((8, ne), lambda i: (0, 0)),
        ],
        out_shape=[
            jax.ShapeDtypeStruct((ntok, 8), jnp.float32),
            jax.ShapeDtypeStruct((8, ne), jnp.float32),
        ],
        scratch_shapes=[pltpu.VMEM((8, ne), jnp.float32)],
        compiler_params=pltpu.CompilerParams(
            dimension_semantics=("arbitrary",)),
    )(input, wg, gumbel)

    combine_weights, dispatch_mask, laux = pl.pallas_call(
        functools.partial(_pass2_kernel, bt=bt, ne=ne, cap=cap, ntok=ntok),
        grid=(nblocks,),
        in_specs=[
            pl.BlockSpec((bt, 8), lambda i: (i, 0)),
            pl.BlockSpec((8, ne), lambda i: (0, 0)),
        ],
        out_specs=[
            pl.BlockSpec((bt, ne, cap), lambda i: (i, 0, 0)),
            pl.BlockSpec((bt, ne, cap), lambda i: (i, 0, 0)),
            pl.BlockSpec(memory_space=pltpu.SMEM),
        ],
        out_shape=[
            jax.ShapeDtypeStruct((ntok, ne, cap), jnp.float32),
            jax.ShapeDtypeStruct((ntok, ne, cap), jnp.bool_),
            jax.ShapeDtypeStruct((1, 1), jnp.float32),
        ],
        compiler_params=pltpu.CompilerParams(
            dimension_semantics=("arbitrary",)),
    )(tok, stats)

    return laux[0, 0], combine_weights, dispatch_mask


# speedup vs baseline: 2.8425x; 2.8425x over previous
import functools

import jax
import jax.numpy as jnp
from jax.experimental import pallas as pl
from jax.experimental.pallas import tpu as pltpu


def _pass1_kernel(x_ref, wg_ref, gum_ref, tok_ref, stats_ref, carry_ref, *,
                  nblocks, bt, ne):
    i = pl.program_id(0)

    @pl.when(i == 0)
    def _init():
        carry_ref[...] = jnp.zeros_like(carry_ref)

    x = x_ref[...]
    wg = wg_ref[...]
    logits = jax.lax.dot_general(
        x, wg, dimension_numbers=(((1,), (1,)), ((), ())),
        preferred_element_type=jnp.float32)

    m = jnp.max(logits, axis=1, keepdims=True)
    p = jnp.exp(logits - m)
    gates = p / jnp.sum(p, axis=1, keepdims=True)

    iota_e = jax.lax.broadcasted_iota(jnp.int32, (bt, ne), 1)

    gmax = jnp.max(gates, axis=1, keepdims=True)
    idx1 = jnp.min(jnp.where(gates == gmax, iota_e, ne), axis=1)
    mask1 = iota_e == idx1[:, None]
    mask1_f = mask1.astype(jnp.float32)

    lw = logits + gum_ref[...]
    lx = jnp.where(mask1, -jnp.inf, lw)
    m2 = jnp.max(lx, axis=1, keepdims=True)
    idx2 = jnp.min(jnp.where(lx == m2, iota_e, ne), axis=1)
    mask2 = iota_e == idx2[:, None]
    mask2_f = mask2.astype(jnp.float32)

    r = jax.lax.broadcasted_iota(jnp.int32, (bt, bt), 0)
    c = jax.lax.broadcasted_iota(jnp.int32, (bt, bt), 1)
    tri = (r >= c).astype(jnp.float32)
    c1 = jax.lax.dot_general(tri, mask1_f,
                             dimension_numbers=(((1,), (0,)), ((), ())),
                             preferred_element_type=jnp.float32)
    c2 = jax.lax.dot_general(tri, mask2_f,
                             dimension_numbers=(((1,), (0,)), ((), ())),
                             preferred_element_type=jnp.float32)

    carry1 = carry_ref[0:1, :]
    carry2 = carry_ref[1:2, :]

    loc1_s = jnp.sum((c1 - 1.0 + carry1) * mask1_f, axis=1)
    loc2p_s = jnp.sum((c2 - 1.0 + carry2) * mask2_f, axis=1)

    g1 = jnp.sum(gates * mask1_f, axis=1)
    g2 = jnp.sum(gates * mask2_f, axis=1)

    carry_ref[0:1, :] = carry1 + jnp.sum(mask1_f, axis=0, keepdims=True)
    carry_ref[1:2, :] = carry2 + jnp.sum(mask2_f, axis=0, keepdims=True)
    carry_ref[2:3, :] = carry_ref[2:3, :] + jnp.sum(gates, axis=0,
                                                    keepdims=True)

    e1f = idx1.astype(jnp.float32)
    e2f = idx2.astype(jnp.float32)
    zeros = jnp.zeros((bt,), jnp.float32)
    tok = jnp.stack([e1f, e2f, g1, g2, loc1_s, loc2p_s, zeros, zeros], axis=1)
    tok_ref[...] = tok

    @pl.when(i == nblocks - 1)
    def _emit_stats():
        stats_ref[...] = carry_ref[...]


def _pass2_kernel(tokt_ref, statst_ref, comb_ref, disp_ref, laux_ref, *,
                  ne, cap, ntok, eb):
    i = pl.program_id(0)
    t = tokt_ref[...]
    e1 = t[0:1, :]
    e2 = t[1:2, :]
    g1 = t[2:3, :]
    g2 = t[3:4, :]
    loc1 = t[4:5, :]
    loc2p = t[5:6, :]

    cnt1c = statst_ref[:, 0:1]
    sub_iota = jax.lax.broadcasted_iota(jnp.int32, (ne, ntok), 0)
    oh2 = sub_iota == e2.astype(jnp.int32)
    addv = jnp.sum(jnp.where(oh2, cnt1c, 0.0), axis=0, keepdims=True)
    loc2 = loc2p + addv

    capf = jnp.float32(cap)
    keep1 = loc1 < capf
    keep2 = loc2 < capf
    g1e = jnp.where(keep1, g1, 0.0)
    g2e = jnp.where(keep2, g2, 0.0)
    denom = jnp.maximum(g1e + g2e, jnp.finfo(jnp.float32).eps)
    g1n = g1e / denom
    g2n = g2e / denom

    loc1s = jnp.where(keep1, loc1, 0.0)
    loc2s = jnp.where(keep2, loc2, 0.0)

    base = jnp.float32(eb * cap) * i.astype(jnp.float32)
    f1 = (e1 * capf + loc1s - base).astype(jnp.int32)
    f2 = (e2 * capf + loc2s - base).astype(jnp.int32)

    rows = eb * cap
    row_iota = jax.lax.broadcasted_iota(jnp.int32, (rows, ntok), 0)
    m1 = row_iota == f1
    m2 = row_iota == f2
    comb = jnp.where(m1, g1n, 0.0) + jnp.where(m2, g2n, 0.0)
    comb_ref[...] = comb.reshape(eb, cap, ntok)
    disp_ref[...] = (comb != 0.0).astype(jnp.int8).reshape(eb, cap, ntok)

    @pl.when(i == 0)
    def _emit_laux():
        me_ce = statst_ref[:, 2:3] * statst_ref[:, 0:1]
        laux_ref[0, 0] = (jnp.sum(me_ce) * jnp.float32(ne)
                          / jnp.float32(ntok) / jnp.float32(ntok))


def kernel(input, wg):
    ntok, dim = input.shape
    ne = wg.shape[0]
    cap = 2 * ntok // ne
    bt = 256
    nblocks = ntok // bt

    gumbel = jax.random.gumbel(jax.random.key(42), (ntok, ne),
                               dtype=jnp.float32)

    tok, stats = pl.pallas_call(
        functools.partial(_pass1_kernel, nblocks=nblocks, bt=bt, ne=ne),
        grid=(nblocks,),
        in_specs=[
            pl.BlockSpec((bt, dim), lambda i: (i, 0)),
            pl.BlockSpec((ne, dim), lambda i: (0, 0)),
            pl.BlockSpec((bt, ne), lambda i: (i, 0)),
        ],
        out_specs=[
            pl.BlockSpec((bt, 8), lambda i: (i, 0)),
            pl.BlockSpec((8, ne), lambda i: (0, 0)),
        ],
        out_shape=[
            jax.ShapeDtypeStruct((ntok, 8), jnp.float32),
            jax.ShapeDtypeStruct((8, ne), jnp.float32),
        ],
        scratch_shapes=[pltpu.VMEM((8, ne), jnp.float32)],
        compiler_params=pltpu.CompilerParams(
            dimension_semantics=("arbitrary",)),
    )(input, wg, gumbel)

    tokt = tok.T
    statst = stats.T

    eb = 8
    neb = ne // eb

    comb_t, disp_t, laux = pl.pallas_call(
        functools.partial(_pass2_kernel, ne=ne, cap=cap, ntok=ntok, eb=eb),
        grid=(neb,),
        in_specs=[
            pl.BlockSpec((8, ntok), lambda i: (0, 0)),
            pl.BlockSpec((ne, 8), lambda i: (0, 0)),
        ],
        out_specs=[
            pl.BlockSpec((eb, cap, ntok), lambda i: (i, 0, 0)),
            pl.BlockSpec((eb, cap, ntok), lambda i: (i, 0, 0)),
            pl.BlockSpec(memory_space=pltpu.SMEM),
        ],
        out_shape=[
            jax.ShapeDtypeStruct((ne, cap, ntok), jnp.float32),
            jax.ShapeDtypeStruct((ne, cap, ntok), jnp.int8),
            jax.ShapeDtypeStruct((1, 1), jnp.float32),
        ],
        compiler_params=pltpu.CompilerParams(
            dimension_semantics=("arbitrary",)),
    )(tokt, statst)

    combine_weights = jnp.transpose(comb_t, (2, 0, 1))
    dispatch_mask = jnp.transpose(disp_t, (2, 0, 1)).astype(bool)
    return laux[0, 0], combine_weights, dispatch_mask


# speedup vs baseline: 3.0083x; 1.0583x over previous
import functools

import jax
import jax.numpy as jnp
from jax.experimental import pallas as pl
from jax.experimental.pallas import tpu as pltpu


def _pass1_kernel(x_ref, wg_ref, gum_ref, tok_ref, stats_ref, carry_ref, *,
                  nblocks, bt, ne):
    i = pl.program_id(0)

    @pl.when(i == 0)
    def _init():
        carry_ref[...] = jnp.zeros_like(carry_ref)

    x = x_ref[...]
    wg = wg_ref[...]
    logits = jax.lax.dot_general(
        x, wg, dimension_numbers=(((1,), (1,)), ((), ())),
        preferred_element_type=jnp.float32)

    m = jnp.max(logits, axis=1, keepdims=True)
    p = jnp.exp(logits - m)
    gates = p / jnp.sum(p, axis=1, keepdims=True)

    iota_e = jax.lax.broadcasted_iota(jnp.int32, (bt, ne), 1)

    gmax = jnp.max(gates, axis=1, keepdims=True)
    idx1 = jnp.min(jnp.where(gates == gmax, iota_e, ne), axis=1)
    mask1 = iota_e == idx1[:, None]
    mask1_f = mask1.astype(jnp.float32)

    lw = logits + gum_ref[...]
    lx = jnp.where(mask1, -jnp.inf, lw)
    m2 = jnp.max(lx, axis=1, keepdims=True)
    idx2 = jnp.min(jnp.where(lx == m2, iota_e, ne), axis=1)
    mask2 = iota_e == idx2[:, None]
    mask2_f = mask2.astype(jnp.float32)

    r = jax.lax.broadcasted_iota(jnp.int32, (bt, bt), 0)
    c = jax.lax.broadcasted_iota(jnp.int32, (bt, bt), 1)
    tri = (r >= c).astype(jnp.float32)
    c1 = jax.lax.dot_general(tri, mask1_f,
                             dimension_numbers=(((1,), (0,)), ((), ())),
                             preferred_element_type=jnp.float32)
    c2 = jax.lax.dot_general(tri, mask2_f,
                             dimension_numbers=(((1,), (0,)), ((), ())),
                             preferred_element_type=jnp.float32)

    carry1 = carry_ref[0:1, :]
    carry2 = carry_ref[1:2, :]

    loc1_s = jnp.sum((c1 - 1.0 + carry1) * mask1_f, axis=1)
    loc2p_s = jnp.sum((c2 - 1.0 + carry2) * mask2_f, axis=1)

    g1 = jnp.sum(gates * mask1_f, axis=1)
    g2 = jnp.sum(gates * mask2_f, axis=1)

    carry_ref[0:1, :] = carry1 + jnp.sum(mask1_f, axis=0, keepdims=True)
    carry_ref[1:2, :] = carry2 + jnp.sum(mask2_f, axis=0, keepdims=True)
    carry_ref[2:3, :] = carry_ref[2:3, :] + jnp.sum(gates, axis=0,
                                                    keepdims=True)

    e1f = idx1.astype(jnp.float32)
    e2f = idx2.astype(jnp.float32)
    zeros = jnp.zeros((bt,), jnp.float32)
    tok = jnp.stack([e1f, e2f, g1, g2, loc1_s, loc2p_s, zeros, zeros], axis=0)
    tok_ref[...] = tok

    @pl.when(i == nblocks - 1)
    def _emit_stats():
        stats_ref[...] = carry_ref[...].T


def _pass2_kernel(tokt_ref, statst_ref, comb_ref, disp_ref, laux_ref, *,
                  ne, cap, ntok, eb):
    i = pl.program_id(0)
    t = tokt_ref[...]
    e1 = t[0:1, :]
    e2 = t[1:2, :]
    g1 = t[2:3, :]
    g2 = t[3:4, :]
    loc1 = t[4:5, :]
    loc2p = t[5:6, :]

    cnt1c = statst_ref[:, 0:1]
    sub_iota = jax.lax.broadcasted_iota(jnp.int32, (ne, ntok), 0)
    oh2 = sub_iota == e2.astype(jnp.int32)
    addv = jnp.sum(jnp.where(oh2, cnt1c, 0.0), axis=0, keepdims=True)
    loc2 = loc2p + addv

    capf = jnp.float32(cap)
    keep1 = loc1 < capf
    keep2 = loc2 < capf
    g1e = jnp.where(keep1, g1, 0.0)
    g2e = jnp.where(keep2, g2, 0.0)
    denom = jnp.maximum(g1e + g2e, jnp.finfo(jnp.float32).eps)
    g1n = g1e / denom
    g2n = g2e / denom

    loc1s = jnp.where(keep1, loc1, 0.0)
    loc2s = jnp.where(keep2, loc2, 0.0)

    base = jnp.float32(eb * cap) * i.astype(jnp.float32)
    f1 = (e1 * capf + loc1s - base).astype(jnp.int32)
    f2 = (e2 * capf + loc2s - base).astype(jnp.int32)

    rows = eb * cap
    row_iota = jax.lax.broadcasted_iota(jnp.int32, (rows, ntok), 0)
    m1 = row_iota == f1
    m2 = row_iota == f2
    comb = jnp.where(m1, g1n, 0.0) + jnp.where(m2, g2n, 0.0)
    comb_ref[...] = comb.reshape(eb, cap, ntok)
    disp_ref[...] = (comb != 0.0).astype(jnp.int8).reshape(eb, cap, ntok)

    @pl.when(i == 0)
    def _emit_laux():
        me_ce = statst_ref[:, 2:3] * statst_ref[:, 0:1]
        laux_ref[0, 0] = (jnp.sum(me_ce) * jnp.float32(ne)
                          / jnp.float32(ntok) / jnp.float32(ntok))


def kernel(input, wg):
    ntok, dim = input.shape
    ne = wg.shape[0]
    cap = 2 * ntok // ne
    bt = 256
    nblocks = ntok // bt

    gumbel = jax.random.gumbel(jax.random.key(42), (ntok, ne),
                               dtype=jnp.float32)

    tok, stats = pl.pallas_call(
        functools.partial(_pass1_kernel, nblocks=nblocks, bt=bt, ne=ne),
        grid=(nblocks,),
        in_specs=[
            pl.BlockSpec((bt, dim), lambda i: (i, 0)),
            pl.BlockSpec((ne, dim), lambda i: (0, 0)),
            pl.BlockSpec((bt, ne), lambda i: (i, 0)),
        ],
        out_specs=[
            pl.BlockSpec((8, bt), lambda i: (0, i)),
            pl.BlockSpec((ne, 8), lambda i: (0, 0)),
        ],
        out_shape=[
            jax.ShapeDtypeStruct((8, ntok), jnp.float32),
            jax.ShapeDtypeStruct((ne, 8), jnp.float32),
        ],
        scratch_shapes=[pltpu.VMEM((8, ne), jnp.float32)],
        compiler_params=pltpu.CompilerParams(
            dimension_semantics=("arbitrary",)),
    )(input, wg, gumbel)

    tokt = tok
    statst = stats

    eb = 8
    neb = ne // eb

    comb_t, disp_t, laux = pl.pallas_call(
        functools.partial(_pass2_kernel, ne=ne, cap=cap, ntok=ntok, eb=eb),
        grid=(neb,),
        in_specs=[
            pl.BlockSpec((8, ntok), lambda i: (0, 0)),
            pl.BlockSpec((ne, 8), lambda i: (0, 0)),
        ],
        out_specs=[
            pl.BlockSpec((eb, cap, ntok), lambda i: (i, 0, 0)),
            pl.BlockSpec((eb, cap, ntok), lambda i: (i, 0, 0)),
            pl.BlockSpec(memory_space=pltpu.SMEM),
        ],
        out_shape=[
            jax.ShapeDtypeStruct((ne, cap, ntok), jnp.float32),
            jax.ShapeDtypeStruct((ne, cap, ntok), jnp.int8),
            jax.ShapeDtypeStruct((1, 1), jnp.float32),
        ],
        compiler_params=pltpu.CompilerParams(
            dimension_semantics=("arbitrary",)),
    )(tokt, statst)

    combine_weights = jnp.transpose(comb_t, (2, 0, 1))
    dispatch_mask = jnp.transpose(disp_t, (2, 0, 1)).astype(bool)
    return laux[0, 0], combine_weights, dispatch_mask


# speedup vs baseline: 3.0510x; 1.0142x over previous
import functools

import jax
import jax.numpy as jnp
from jax.experimental import pallas as pl
from jax.experimental.pallas import tpu as pltpu


def _pass1_kernel(x_ref, wg_ref, gum_ref, tok_ref, stats_ref, carry_ref, *,
                  nblocks, bt, ne):
    i = pl.program_id(0)

    @pl.when(i == 0)
    def _init():
        carry_ref[...] = jnp.zeros_like(carry_ref)

    x = x_ref[...]
    wg = wg_ref[...]
    logits = jax.lax.dot_general(
        x, wg, dimension_numbers=(((1,), (1,)), ((), ())),
        preferred_element_type=jnp.float32)

    m = jnp.max(logits, axis=1, keepdims=True)
    p = jnp.exp(logits - m)
    gates = p / jnp.sum(p, axis=1, keepdims=True)

    iota_e = jax.lax.broadcasted_iota(jnp.int32, (bt, ne), 1)

    gmax = jnp.max(gates, axis=1, keepdims=True)
    idx1 = jnp.min(jnp.where(gates == gmax, iota_e, ne), axis=1)
    mask1 = iota_e == idx1[:, None]
    mask1_f = mask1.astype(jnp.float32)

    lw = logits + gum_ref[...]
    lx = jnp.where(mask1, -jnp.inf, lw)
    m2 = jnp.max(lx, axis=1, keepdims=True)
    idx2 = jnp.min(jnp.where(lx == m2, iota_e, ne), axis=1)
    mask2 = iota_e == idx2[:, None]
    mask2_f = mask2.astype(jnp.float32)

    r = jax.lax.broadcasted_iota(jnp.int32, (bt, bt), 0)
    c = jax.lax.broadcasted_iota(jnp.int32, (bt, bt), 1)
    tri = (r >= c).astype(jnp.float32)
    c1 = jax.lax.dot_general(tri, mask1_f,
                             dimension_numbers=(((1,), (0,)), ((), ())),
                             preferred_element_type=jnp.float32)
    c2 = jax.lax.dot_general(tri, mask2_f,
                             dimension_numbers=(((1,), (0,)), ((), ())),
                             preferred_element_type=jnp.float32)

    carry1 = carry_ref[0:1, :]
    carry2 = carry_ref[1:2, :]

    loc1_s = jnp.sum((c1 - 1.0 + carry1) * mask1_f, axis=1)
    loc2p_s = jnp.sum((c2 - 1.0 + carry2) * mask2_f, axis=1)

    g1 = jnp.sum(gates * mask1_f, axis=1)
    g2 = jnp.sum(gates * mask2_f, axis=1)

    carry_ref[0:1, :] = carry1 + jnp.sum(mask1_f, axis=0, keepdims=True)
    carry_ref[1:2, :] = carry2 + jnp.sum(mask2_f, axis=0, keepdims=True)
    carry_ref[2:3, :] = carry_ref[2:3, :] + jnp.sum(gates, axis=0,
                                                    keepdims=True)

    e1f = idx1.astype(jnp.float32)
    e2f = idx2.astype(jnp.float32)
    zeros = jnp.zeros((bt,), jnp.float32)
    tok = jnp.stack([e1f, e2f, g1, g2, loc1_s, loc2p_s, zeros, zeros], axis=0)
    tok_ref[...] = tok

    @pl.when(i == nblocks - 1)
    def _emit_stats():
        stats_ref[...] = carry_ref[...].T


def _pass2_kernel(tokt_ref, statst_ref, comb_ref, disp_ref, laux_ref, *,
                  ne, cap, ntok, eb):
    i = pl.program_id(0)
    t = tokt_ref[...]
    e1 = t[0:1, :]
    e2 = t[1:2, :]
    g1 = t[2:3, :]
    g2 = t[3:4, :]
    loc1 = t[4:5, :]
    loc2p = t[5:6, :]

    cnt1c = statst_ref[:, 0:1]
    sub_iota = jax.lax.broadcasted_iota(jnp.int32, (ne, ntok), 0)
    oh2 = sub_iota == e2.astype(jnp.int32)
    addv = jnp.sum(jnp.where(oh2, cnt1c, 0.0), axis=0, keepdims=True)
    loc2 = loc2p + addv

    capf = jnp.float32(cap)
    keep1 = loc1 < capf
    keep2 = loc2 < capf
    g1e = jnp.where(keep1, g1, 0.0)
    g2e = jnp.where(keep2, g2, 0.0)
    denom = jnp.maximum(g1e + g2e, jnp.finfo(jnp.float32).eps)
    g1n = g1e / denom
    g2n = g2e / denom

    loc1s = jnp.where(keep1, loc1, 0.0)
    loc2s = jnp.where(keep2, loc2, 0.0)

    base = jnp.float32(eb * cap) * i.astype(jnp.float32)
    f1 = (e1 * capf + loc1s - base).astype(jnp.int32)
    f2 = (e2 * capf + loc2s - base).astype(jnp.int32)

    rows = eb * cap
    row_iota = jax.lax.broadcasted_iota(jnp.int32, (rows, ntok), 0)
    m1 = row_iota == f1
    m2 = row_iota == f2
    comb = jnp.where(m1, g1n, 0.0) + jnp.where(m2, g2n, 0.0)
    comb_ref[...] = comb.reshape(eb, cap, ntok)
    disp_ref[...] = (comb != 0.0).astype(jnp.int8).reshape(eb, cap, ntok)

    @pl.when(i == 0)
    def _emit_laux():
        me_ce = statst_ref[:, 2:3] * statst_ref[:, 0:1]
        laux_ref[0, 0] = (jnp.sum(me_ce) * jnp.float32(ne)
                          / jnp.float32(ntok) / jnp.float32(ntok))


def kernel(input, wg):
    ntok, dim = input.shape
    ne = wg.shape[0]
    cap = 2 * ntok // ne
    bt = 512
    nblocks = ntok // bt

    gumbel = jax.random.gumbel(jax.random.key(42), (ntok, ne),
                               dtype=jnp.float32)

    tok, stats = pl.pallas_call(
        functools.partial(_pass1_kernel, nblocks=nblocks, bt=bt, ne=ne),
        grid=(nblocks,),
        in_specs=[
            pl.BlockSpec((bt, dim), lambda i: (i, 0)),
            pl.BlockSpec((ne, dim), lambda i: (0, 0)),
            pl.BlockSpec((bt, ne), lambda i: (i, 0)),
        ],
        out_specs=[
            pl.BlockSpec((8, bt), lambda i: (0, i)),
            pl.BlockSpec((ne, 8), lambda i: (0, 0)),
        ],
        out_shape=[
            jax.ShapeDtypeStruct((8, ntok), jnp.float32),
            jax.ShapeDtypeStruct((ne, 8), jnp.float32),
        ],
        scratch_shapes=[pltpu.VMEM((8, ne), jnp.float32)],
        compiler_params=pltpu.CompilerParams(
            dimension_semantics=("arbitrary",)),
    )(input, wg, gumbel)

    tokt = tok
    statst = stats

    eb = 16
    neb = ne // eb

    comb_t, disp_t, laux = pl.pallas_call(
        functools.partial(_pass2_kernel, ne=ne, cap=cap, ntok=ntok, eb=eb),
        grid=(neb,),
        in_specs=[
            pl.BlockSpec((8, ntok), lambda i: (0, 0)),
            pl.BlockSpec((ne, 8), lambda i: (0, 0)),
        ],
        out_specs=[
            pl.BlockSpec((eb, cap, ntok), lambda i: (i, 0, 0)),
            pl.BlockSpec((eb, cap, ntok), lambda i: (i, 0, 0)),
            pl.BlockSpec(memory_space=pltpu.SMEM),
        ],
        out_shape=[
            jax.ShapeDtypeStruct((ne, cap, ntok), jnp.float32),
            jax.ShapeDtypeStruct((ne, cap, ntok), jnp.int8),
            jax.ShapeDtypeStruct((1, 1), jnp.float32),
        ],
        compiler_params=pltpu.CompilerParams(
            dimension_semantics=("arbitrary",)),
    )(tokt, statst)

    combine_weights = jnp.transpose(comb_t, (2, 0, 1))
    dispatch_mask = jnp.transpose(disp_t, (2, 0, 1)).astype(bool)
    return laux[0, 0], combine_weights, dispatch_mask
